# baseline (speedup 1.0000x reference)
import functools

import jax
import jax.numpy as jnp
from jax import lax
from jax.experimental import pallas as pl
from jax.experimental.pallas import tpu as pltpu
from jax.experimental.pallas import tpu_sc as plsc

VOCAB = 1000000
DIM = 32
BATCH = 16384
HIST = 50

NC = 2
NS = 16
NW = NC * NS
BW = BATCH // NW
C = 16
NCHUNK = BW // C
ROWS = C * HIST
SUB = 100
NSUB = ROWS // SUB


def _make_kernel():
  mesh = plsc.VectorSubcoreMesh(core_axis_name="c", subcore_axis_name="s")

  @functools.partial(
      pl.kernel,
      mesh=mesh,
      out_type=jax.ShapeDtypeStruct((BATCH, DIM), jnp.float32),
      scratch_types=[
          pltpu.VMEM((NSUB, SUB), jnp.int32),
          pltpu.VMEM((ROWS, DIM), jnp.float32),
          pltpu.VMEM((C, DIM), jnp.float32),
          pltpu.SemaphoreType.DMA,
      ],
      compiler_params=pltpu.CompilerParams(use_tc_tiling_on_sc=False),
  )
  def emb_bag(table_hbm, idxq_hbm, out_hbm, idxq_v, rows_v, out_v, sem):
    wid = lax.axis_index("s") * NC + lax.axis_index("c")
    scale = jnp.float32(1.0 / HIST)

    def chunk_body(c, _):
      pltpu.sync_copy(idxq_hbm.at[wid, c], idxq_v)

      def fire(s, _):
        pltpu.async_copy(
            table_hbm.at[idxq_v.at[s]],
            rows_v.at[pl.ds(s * SUB, SUB)],
            sem,
        )
        return 0

      lax.fori_loop(0, NSUB, fire, 0)
      pltpu.make_async_copy(table_hbm.at[pl.ds(0, ROWS)], rows_v, sem).wait()

      def bag_body(b, _):
        base = b * HIST
        acc0 = rows_v[base, pl.ds(0, 16)]
        acc1 = rows_v[base, pl.ds(16, 16)]
        for l in range(1, HIST):
          acc0 = acc0 + rows_v[base + l, pl.ds(0, 16)]
          acc1 = acc1 + rows_v[base + l, pl.ds(16, 16)]
        out_v[b, pl.ds(0, 16)] = acc0 * scale
        out_v[b, pl.ds(16, 16)] = acc1 * scale
        return 0

      lax.fori_loop(0, C, bag_body, 0)
      pltpu.sync_copy(out_v, out_hbm.at[pl.ds(wid * BW + c * C, C)])
      return 0

    lax.fori_loop(0, NCHUNK, chunk_body, 0)

  return emb_bag


_emb_bag = _make_kernel()


@jax.jit
def kernel(indices, table):
  idxq = indices.astype(jnp.int32).reshape(NW, NCHUNK, NSUB, SUB)
  return _emb_bag(table, idxq)


# speedup vs baseline: 1.1004x; 1.1004x over previous
import functools

import jax
import jax.numpy as jnp
from jax import lax
from jax.experimental import pallas as pl
from jax.experimental.pallas import tpu as pltpu
from jax.experimental.pallas import tpu_sc as plsc

VOCAB = 1000000
DIM = 32
BATCH = 16384
HIST = 50

NC = 2
NS = 16
NW = NC * NS
BW = BATCH // NW
C = 16
NCHUNK = BW // C
NPAIR = NCHUNK // 2
ROWS = C * HIST
SUB = 100
NSUB = ROWS // SUB
NIDX = NCHUNK * NSUB


def _make_kernel():
  mesh = plsc.VectorSubcoreMesh(core_axis_name="c", subcore_axis_name="s")

  @functools.partial(
      pl.kernel,
      mesh=mesh,
      out_type=jax.ShapeDtypeStruct((BATCH, DIM), jnp.float32),
      scratch_types=[
          pltpu.VMEM((NIDX, SUB), jnp.int32),
          pltpu.VMEM((ROWS, DIM), jnp.float32),
          pltpu.VMEM((ROWS, DIM), jnp.float32),
          pltpu.VMEM((C, DIM), jnp.float32),
          pltpu.SemaphoreType.DMA,
          pltpu.SemaphoreType.DMA,
      ],
      compiler_params=pltpu.CompilerParams(use_tc_tiling_on_sc=False),
  )
  def emb_bag(table_hbm, idxq_hbm, out_hbm, idx_v, rows_a, rows_b, out_v,
              sem_a, sem_b):
    wid = lax.axis_index("s") * NC + lax.axis_index("c")
    scale = jnp.float32(1.0 / HIST)

    pltpu.sync_copy(idxq_hbm.at[wid], idx_v)

    def fire(c, rows_v, sem):
      def body(s, _):
        pltpu.async_copy(
            table_hbm.at[idx_v.at[c * NSUB + s]],
            rows_v.at[pl.ds(s * SUB, SUB)],
            sem,
        )
        return 0

      lax.fori_loop(0, NSUB, body, 0)

    def drain(rows_v, sem):
      pltpu.make_async_copy(table_hbm.at[pl.ds(0, ROWS)], rows_v, sem).wait()

    def reduce_store(c, rows_v):
      def bag_body(b, _):
        base = b * HIST
        acc0 = rows_v[base, pl.ds(0, 16)]
        acc1 = rows_v[base, pl.ds(16, 16)]
        for l in range(1, HIST):
          acc0 = acc0 + rows_v[base + l, pl.ds(0, 16)]
          acc1 = acc1 + rows_v[base + l, pl.ds(16, 16)]
        out_v[b, pl.ds(0, 16)] = acc0 * scale
        out_v[b, pl.ds(16, 16)] = acc1 * scale
        return 0

      lax.fori_loop(0, C, bag_body, 0)
      pltpu.sync_copy(out_v, out_hbm.at[pl.ds(wid * BW + c * C, C)])

    fire(0, rows_a, sem_a)

    def pair_body(p, _):
      c0 = 2 * p
      fire(c0 + 1, rows_b, sem_b)
      drain(rows_a, sem_a)
      reduce_store(c0, rows_a)

      @pl.when(p < NPAIR - 1)
      def _():
        fire(c0 + 2, rows_a, sem_a)

      drain(rows_b, sem_b)
      reduce_store(c0 + 1, rows_b)
      return 0

    lax.fori_loop(0, NPAIR, pair_body, 0)

  return emb_bag


_emb_bag = _make_kernel()


@jax.jit
def kernel(indices, table):
  idxq = indices.astype(jnp.int32).reshape(NW, NIDX, SUB)
  return _emb_bag(table, idxq)
